# baseline (speedup 1.0000x reference)
import functools

import jax
import jax.numpy as jnp
from jax import lax
from jax.experimental import pallas as pl
from jax.experimental.pallas import tpu as pltpu
from jax.experimental.pallas import tpu_sc as plsc

V = 1000
BT = 51200
NC, NS = 2, 16
NW = NC * NS
NB = 125
REPS = 4
NCOL = REPS * 8
S = 256
NCHUNK = BT // S
BPW = BT // NW


def _lse_body(table_ref, lse_ref):
    t = table_ref[...]
    m = jnp.max(t, axis=1)
    s = jnp.sum(jnp.exp(t - m[:, None]), axis=1)
    lse_ref[...] = m + jnp.log(s)


def _lse_call(table):
    return pl.pallas_call(
        _lse_body,
        out_shape=jax.ShapeDtypeStruct((V,), jnp.float32),
    )(table)


def _loss_body(part_ref, out_ref):
    out_ref[0, 0] = jnp.sum(part_ref[...]) * (1.0 / BT)


def _loss_call(part):
    return pl.pallas_call(
        _loss_body,
        out_shape=jax.ShapeDtypeStruct((1, 1), jnp.float32),
        out_specs=pl.BlockSpec(memory_space=pltpu.SMEM),
    )(part)


@functools.partial(
    pl.kernel,
    out_type=(
        jax.ShapeDtypeStruct((NB, 8, BT), jnp.float32),
        jax.ShapeDtypeStruct((NW, 16), jnp.float32),
    ),
    mesh=plsc.VectorSubcoreMesh(core_axis_name="c", subcore_axis_name="s"),
    compiler_params=pltpu.CompilerParams(
        needs_layout_passes=False, use_tc_tiling_on_sc=True
    ),
    scratch_types=[
        pltpu.VMEM((NCOL * V,), jnp.float32),
        pltpu.VMEM((2, S), jnp.int32),
        pltpu.VMEM((2, S), jnp.int32),
        pltpu.VMEM((V,), jnp.float32),
        pltpu.VMEM((BPW,), jnp.int32),
        pltpu.VMEM((2, NCOL, S), jnp.float32),
        pltpu.VMEM((16,), jnp.float32),
        pltpu.SemaphoreType.DMA,
        pltpu.SemaphoreType.DMA,
        pltpu.SemaphoreType.DMA,
        pltpu.SemaphoreType.DMA,
    ],
)
def _sc_gather(tt_hbm, idx_hbm, tgt_hbm, lse_hbm, out_hbm, part_hbm,
               tt_v, idx_v, tgt_v, lse_v, idxl_v, slab_v, acc_v,
               lsem0, lsem1, wsem0, wsem1):
    wid = lax.axis_index("s") * NC + lax.axis_index("c")

    for rep in range(REPS):
        b = wid + NW * rep

        @pl.when(b < NB)
        def _():
            pltpu.sync_copy(tt_hbm.at[pl.ds(b * 8 * V, 8 * V)],
                            tt_v.at[pl.ds(rep * 8 * V, 8 * V)])

    pltpu.sync_copy(lse_hbm, lse_v)

    acc_v[...] = jnp.zeros((16,), jnp.float32)
    pltpu.sync_copy(idx_hbm.at[pl.ds(wid * BPW, BPW)], idxl_v)

    def lse_grp(g):
        tok = idxl_v[pl.ds(g * 16, 16)]
        plsc.addupdate(acc_v.at[:], plsc.load_gather(lse_v, [tok]))

    plsc.parallel_loop(0, BPW // 16, 1, unroll=8)(lse_grp)

    def load_chunk(ci, par, lsem):
        pltpu.async_copy(idx_hbm.at[pl.ds(ci * S, S)], idx_v.at[par], lsem)
        pltpu.async_copy(tgt_hbm.at[pl.ds(ci * S, S)], tgt_v.at[par], lsem)

    def wait_load(par, lsem):
        pltpu.make_async_copy(idx_hbm.at[pl.ds(0, S)], idx_v.at[par],
                              lsem).wait()
        pltpu.make_async_copy(tgt_hbm.at[pl.ds(0, S)], tgt_v.at[par],
                              lsem).wait()

    def start_writes(ci, par, wsem):
        s0 = ci * S
        for rep in range(REPS):
            b = wid + NW * rep

            @pl.when(b < NB)
            def _():
                pltpu.async_copy(slab_v.at[par, pl.ds(rep * 8, 8)],
                                 out_hbm.at[b, :, pl.ds(s0, S)], wsem)

    def wait_writes(par, wsem):
        for rep in range(REPS):
            b = wid + NW * rep

            @pl.when(b < NB)
            def _():
                pltpu.make_async_copy(slab_v.at[par, pl.ds(rep * 8, 8)],
                                      out_hbm.at[b, :, pl.ds(0, S)],
                                      wsem).wait()

    load_chunk(0, 0, lsem0)
    load_chunk(1, 1, lsem1)

    def chunk(ci, carry):
        par = lax.rem(ci, 2)

        @pl.when(par == 0)
        def _():
            wait_load(0, lsem0)

        @pl.when(par == 1)
        def _():
            wait_load(1, lsem1)

        @pl.when((par == 0) & (ci >= 2))
        def _():
            wait_writes(0, wsem0)

        @pl.when((par == 1) & (ci >= 2))
        def _():
            wait_writes(1, wsem1)

        for gi in range(S // 16):
            gb = gi * 16
            tok = idx_v[par, pl.ds(gb, 16)]
            tg = tgt_v[par, pl.ds(gb, 16)]

            def col4(k0):
                for j in range(4):
                    k = k0 + j
                    vals = plsc.load_gather(tt_v, [tok + k * V])
                    slab_v[par, k, pl.ds(gb, 16)] = vals

            plsc.parallel_loop(0, NCOL, 4, unroll=8)(col4)

            for rep in range(REPS):
                rb = 8 * (wid + NW * rep)
                m = (tg >= rb) & (tg < rb + 8)
                pidx = (tg - rb + 8 * rep) * V + tok
                p = plsc.load_gather(tt_v, [pidx], mask=m)
                plsc.addupdate(acc_v.at[:], -jnp.where(m, p, 0.0))

        @pl.when(par == 0)
        def _():
            start_writes(ci, 0, wsem0)

            @pl.when(ci + 2 < NCHUNK)
            def _():
                load_chunk(ci + 2, 0, lsem0)

        @pl.when(par == 1)
        def _():
            start_writes(ci, 1, wsem1)

            @pl.when(ci + 2 < NCHUNK)
            def _():
                load_chunk(ci + 2, 1, lsem1)

        return carry

    lax.fori_loop(0, NCHUNK, chunk, jnp.int32(0))
    wait_writes(0, wsem0)
    wait_writes(1, wsem1)
    pltpu.sync_copy(acc_v, part_hbm.at[wid])


def kernel(idx, targets, table):
    idxf = idx.reshape(-1).astype(jnp.int32)
    tgtf = targets.reshape(-1).astype(jnp.int32)
    tt_flat = jnp.transpose(table).reshape(-1)
    lse = _lse_call(table)
    out3, part = _sc_gather(tt_flat, idxf, tgtf, lse)
    logits = jnp.transpose(out3.reshape(V, BT))
    loss = _loss_call(part)[0, 0]
    return (logits, loss)


# speedup vs baseline: 1.4184x; 1.4184x over previous
import functools

import jax
import jax.numpy as jnp
from jax import lax
from jax.experimental import pallas as pl
from jax.experimental.pallas import tpu as pltpu
from jax.experimental.pallas import tpu_sc as plsc

V = 1000
BT = 51200
NC, NS = 2, 16
NW = NC * NS
NB = 125
REPS = 4
NCOL = REPS * 8
S = 256
NCHUNK = BT // S
BPW = BT // NW


def _lse_body(table_ref, lse_ref):
    t = table_ref[...]
    m = jnp.max(t, axis=1)
    s = jnp.sum(jnp.exp(t - m[:, None]), axis=1)
    lse_ref[...] = m + jnp.log(s)


def _lse_call(table):
    return pl.pallas_call(
        _lse_body,
        out_shape=jax.ShapeDtypeStruct((V,), jnp.float32),
    )(table)


def _loss_body(part_ref, out_ref):
    out_ref[0, 0] = jnp.sum(part_ref[...]) * (1.0 / BT)


def _loss_call(part):
    return pl.pallas_call(
        _loss_body,
        out_shape=jax.ShapeDtypeStruct((1, 1), jnp.float32),
        out_specs=pl.BlockSpec(memory_space=pltpu.SMEM),
    )(part)


@functools.partial(
    pl.kernel,
    out_type=(
        jax.ShapeDtypeStruct((NB, 8, BT), jnp.float32),
        jax.ShapeDtypeStruct((NW, 16), jnp.float32),
    ),
    mesh=plsc.VectorSubcoreMesh(core_axis_name="c", subcore_axis_name="s"),
    compiler_params=pltpu.CompilerParams(
        needs_layout_passes=False, use_tc_tiling_on_sc=True
    ),
    scratch_types=[
        pltpu.VMEM((NCOL * V,), jnp.float32),
        pltpu.VMEM((2, S), jnp.int32),
        pltpu.VMEM((2, S), jnp.int32),
        pltpu.VMEM((V,), jnp.float32),
        pltpu.VMEM((BPW,), jnp.int32),
        pltpu.VMEM((2, NCOL, S), jnp.float32),
        pltpu.VMEM((16,), jnp.float32),
        pltpu.SemaphoreType.DMA,
        pltpu.SemaphoreType.DMA,
        pltpu.SemaphoreType.DMA,
        pltpu.SemaphoreType.DMA,
    ],
)
def _sc_gather(tt_hbm, idx_hbm, tgt_hbm, lse_hbm, out_hbm, part_hbm,
               tt_v, idx_v, tgt_v, lse_v, idxl_v, slab_v, acc_v,
               lsem0, lsem1, wsem0, wsem1):
    wid = lax.axis_index("s") * NC + lax.axis_index("c")

    for rep in range(REPS):
        b = wid + NW * rep

        @pl.when(b < NB)
        def _():
            pltpu.sync_copy(tt_hbm.at[pl.ds(b * 8 * V, 8 * V)],
                            tt_v.at[pl.ds(rep * 8 * V, 8 * V)])

    pltpu.sync_copy(lse_hbm, lse_v)

    pltpu.sync_copy(idx_hbm.at[pl.ds(wid * BPW, BPW)], idxl_v)

    def lse_grp(g, a):
        tok = idxl_v[pl.ds(g * 16, 16)]
        return a + plsc.load_gather(lse_v, [tok])

    acc = plsc.parallel_loop(0, BPW // 16, 1, unroll=8,
                             carry=jnp.zeros((16,), jnp.float32))(lse_grp)

    def load_chunk(ci, par, lsem):
        pltpu.async_copy(idx_hbm.at[pl.ds(ci * S, S)], idx_v.at[par], lsem)
        pltpu.async_copy(tgt_hbm.at[pl.ds(ci * S, S)], tgt_v.at[par], lsem)

    def wait_load(par, lsem):
        pltpu.make_async_copy(idx_hbm.at[pl.ds(0, S)], idx_v.at[par],
                              lsem).wait()
        pltpu.make_async_copy(tgt_hbm.at[pl.ds(0, S)], tgt_v.at[par],
                              lsem).wait()

    def start_writes(ci, par, wsem):
        s0 = ci * S
        for rep in range(REPS):
            b = wid + NW * rep

            @pl.when(b < NB)
            def _():
                pltpu.async_copy(slab_v.at[par, pl.ds(rep * 8, 8)],
                                 out_hbm.at[b, :, pl.ds(s0, S)], wsem)

    def wait_writes(par, wsem):
        for rep in range(REPS):
            b = wid + NW * rep

            @pl.when(b < NB)
            def _():
                pltpu.make_async_copy(slab_v.at[par, pl.ds(rep * 8, 8)],
                                      out_hbm.at[b, :, pl.ds(0, S)],
                                      wsem).wait()

    load_chunk(0, 0, lsem0)
    load_chunk(1, 1, lsem1)

    def chunk(ci, acc):
        par = lax.rem(ci, 2)

        @pl.when(par == 0)
        def _():
            wait_load(0, lsem0)

        @pl.when(par == 1)
        def _():
            wait_load(1, lsem1)

        @pl.when((par == 0) & (ci >= 2))
        def _():
            wait_writes(0, wsem0)

        @pl.when((par == 1) & (ci >= 2))
        def _():
            wait_writes(1, wsem1)

        def gi_body(gb, acc):
            tok = idx_v[par, pl.ds(gb, 16)]
            tg = tgt_v[par, pl.ds(gb, 16)]

            def col4(k0):
                for j in range(4):
                    k = k0 + j
                    vals = plsc.load_gather(tt_v, [tok + k * V])
                    slab_v[par, k, pl.ds(gb, 16)] = vals

            plsc.parallel_loop(0, NCOL, 4, unroll=8)(col4)

            for rep in range(REPS):
                rb = 8 * (wid + NW * rep)
                m = (tg >= rb) & (tg < rb + 8)
                pidx = (tg - rb + 8 * rep) * V + tok
                p = plsc.load_gather(tt_v, [pidx], mask=m)
                acc = acc - jnp.where(m, p, 0.0)
            return acc

        acc = plsc.parallel_loop(0, S, 16, unroll=2, carry=acc)(gi_body)

        @pl.when(par == 0)
        def _():
            start_writes(ci, 0, wsem0)

            @pl.when(ci + 2 < NCHUNK)
            def _():
                load_chunk(ci + 2, 0, lsem0)

        @pl.when(par == 1)
        def _():
            start_writes(ci, 1, wsem1)

            @pl.when(ci + 2 < NCHUNK)
            def _():
                load_chunk(ci + 2, 1, lsem1)

        return acc

    acc = lax.fori_loop(0, NCHUNK, chunk, acc)
    wait_writes(0, wsem0)
    wait_writes(1, wsem1)
    acc_v[...] = acc
    pltpu.sync_copy(acc_v, part_hbm.at[wid])


def kernel(idx, targets, table):
    idxf = idx.reshape(-1).astype(jnp.int32)
    tgtf = targets.reshape(-1).astype(jnp.int32)
    tt_flat = jnp.transpose(table).reshape(-1)
    lse = _lse_call(table)
    out3, part = _sc_gather(tt_flat, idxf, tgtf, lse)
    logits = jnp.transpose(out3.reshape(V, BT))
    loss = _loss_call(part)[0, 0]
    return (logits, loss)


# speedup vs baseline: 1.5245x; 1.0748x over previous
import functools

import jax
import jax.numpy as jnp
from jax import lax
from jax.experimental import pallas as pl
from jax.experimental.pallas import tpu as pltpu
from jax.experimental.pallas import tpu_sc as plsc

V = 1000
BT = 51200
NC, NS = 2, 16
NW = NC * NS
NB = 125
REPS = 4
NCOL = REPS * 8
S = 512
NCHUNK = BT // S
BPW = BT // NW


def _lse_body(table_ref, lse_ref):
    t = table_ref[...]
    m = jnp.max(t, axis=1)
    s = jnp.sum(jnp.exp(t - m[:, None]), axis=1)
    lse_ref[...] = m + jnp.log(s)


def _lse_call(table):
    return pl.pallas_call(
        _lse_body,
        out_shape=jax.ShapeDtypeStruct((V,), jnp.float32),
    )(table)


def _loss_body(part_ref, out_ref):
    out_ref[0, 0] = jnp.sum(part_ref[...]) * (1.0 / BT)


def _loss_call(part):
    return pl.pallas_call(
        _loss_body,
        out_shape=jax.ShapeDtypeStruct((1, 1), jnp.float32),
        out_specs=pl.BlockSpec(memory_space=pltpu.SMEM),
    )(part)


@functools.partial(
    pl.kernel,
    out_type=(
        jax.ShapeDtypeStruct((NB, 8, BT), jnp.float32),
        jax.ShapeDtypeStruct((NW, 16), jnp.float32),
    ),
    mesh=plsc.VectorSubcoreMesh(core_axis_name="c", subcore_axis_name="s"),
    compiler_params=pltpu.CompilerParams(
        needs_layout_passes=False, use_tc_tiling_on_sc=True
    ),
    scratch_types=[
        pltpu.VMEM((NCOL * V,), jnp.float32),
        pltpu.VMEM((2, S), jnp.int32),
        pltpu.VMEM((2, S), jnp.int32),
        pltpu.VMEM((V,), jnp.float32),
        pltpu.VMEM((BPW,), jnp.int32),
        pltpu.VMEM((2, NCOL, S), jnp.float32),
        pltpu.VMEM((16,), jnp.float32),
        pltpu.SemaphoreType.DMA,
        pltpu.SemaphoreType.DMA,
        pltpu.SemaphoreType.DMA,
        pltpu.SemaphoreType.DMA,
    ],
)
def _sc_gather(tt_hbm, idx_hbm, tgt_hbm, lse_hbm, out_hbm, part_hbm,
               tt_v, idx_v, tgt_v, lse_v, idxl_v, slab_v, acc_v,
               lsem0, lsem1, wsem0, wsem1):
    wid = lax.axis_index("s") * NC + lax.axis_index("c")

    for rep in range(REPS):
        b = wid + NW * rep

        @pl.when(b < NB)
        def _():
            pltpu.sync_copy(tt_hbm.at[pl.ds(b * 8 * V, 8 * V)],
                            tt_v.at[pl.ds(rep * 8 * V, 8 * V)])

    pltpu.sync_copy(lse_hbm, lse_v)

    pltpu.sync_copy(idx_hbm.at[pl.ds(wid * BPW, BPW)], idxl_v)

    def lse_grp(g, a):
        tok = idxl_v[pl.ds(g * 16, 16)]
        return a + plsc.load_gather(lse_v, [tok])

    acc = plsc.parallel_loop(0, BPW // 16, 1, unroll=8,
                             carry=jnp.zeros((16,), jnp.float32))(lse_grp)

    def load_chunk(ci, par, lsem):
        pltpu.async_copy(idx_hbm.at[pl.ds(ci * S, S)], idx_v.at[par], lsem)
        pltpu.async_copy(tgt_hbm.at[pl.ds(ci * S, S)], tgt_v.at[par], lsem)

    def wait_load(par, lsem):
        pltpu.make_async_copy(idx_hbm.at[pl.ds(0, S)], idx_v.at[par],
                              lsem).wait()
        pltpu.make_async_copy(tgt_hbm.at[pl.ds(0, S)], tgt_v.at[par],
                              lsem).wait()

    def start_writes(ci, par, wsem):
        s0 = ci * S
        for rep in range(REPS):
            b = wid + NW * rep

            @pl.when(b < NB)
            def _():
                pltpu.async_copy(slab_v.at[par, pl.ds(rep * 8, 8)],
                                 out_hbm.at[b, :, pl.ds(s0, S)], wsem)

    def wait_writes(par, wsem):
        for rep in range(REPS):
            b = wid + NW * rep

            @pl.when(b < NB)
            def _():
                pltpu.make_async_copy(slab_v.at[par, pl.ds(rep * 8, 8)],
                                      out_hbm.at[b, :, pl.ds(0, S)],
                                      wsem).wait()

    load_chunk(0, 0, lsem0)
    load_chunk(1, 1, lsem1)

    def chunk(ci, acc):
        par = lax.rem(ci, 2)

        @pl.when(par == 0)
        def _():
            wait_load(0, lsem0)

        @pl.when(par == 1)
        def _():
            wait_load(1, lsem1)

        @pl.when((par == 0) & (ci >= 2))
        def _():
            wait_writes(0, wsem0)

        @pl.when((par == 1) & (ci >= 2))
        def _():
            wait_writes(1, wsem1)

        def gi_body(gb, acc):
            tok = idx_v[par, pl.ds(gb, 16)]
            tg = tgt_v[par, pl.ds(gb, 16)]

            def col4(k0):
                for j in range(4):
                    k = k0 + j
                    vals = plsc.load_gather(tt_v, [tok + k * V])
                    slab_v[par, k, pl.ds(gb, 16)] = vals

            plsc.parallel_loop(0, NCOL, 4, unroll=8)(col4)

            for rep in range(REPS):
                rb = 8 * (wid + NW * rep)
                m = (tg >= rb) & (tg < rb + 8)
                pidx = (tg - rb + 8 * rep) * V + tok
                p = plsc.load_gather(tt_v, [pidx], mask=m)
                acc = acc - jnp.where(m, p, 0.0)
            return acc

        acc = plsc.parallel_loop(0, S, 16, unroll=2, carry=acc)(gi_body)

        @pl.when(par == 0)
        def _():
            start_writes(ci, 0, wsem0)

            @pl.when(ci + 2 < NCHUNK)
            def _():
                load_chunk(ci + 2, 0, lsem0)

        @pl.when(par == 1)
        def _():
            start_writes(ci, 1, wsem1)

            @pl.when(ci + 2 < NCHUNK)
            def _():
                load_chunk(ci + 2, 1, lsem1)

        return acc

    acc = lax.fori_loop(0, NCHUNK, chunk, acc)
    wait_writes(0, wsem0)
    wait_writes(1, wsem1)
    acc_v[...] = acc
    pltpu.sync_copy(acc_v, part_hbm.at[wid])


def kernel(idx, targets, table):
    idxf = idx.reshape(-1).astype(jnp.int32)
    tgtf = targets.reshape(-1).astype(jnp.int32)
    tt_flat = jnp.transpose(table).reshape(-1)
    lse = _lse_call(table)
    out3, part = _sc_gather(tt_flat, idxf, tgtf, lse)
    logits = jnp.transpose(out3.reshape(V, BT))
    loss = _loss_call(part)[0, 0]
    return (logits, loss)


# speedup vs baseline: 1.5647x; 1.0264x over previous
import functools

import jax
import jax.numpy as jnp
from jax import lax
from jax.experimental import pallas as pl
from jax.experimental.pallas import tpu as pltpu
from jax.experimental.pallas import tpu_sc as plsc

V = 1000
BT = 51200
NC, NS = 2, 16
NW = NC * NS
NB = 125
REPS = 4
NCOL = REPS * 8
S = 1024
NCHUNK = BT // S
BPW = BT // NW


def _lse_body(table_ref, lse_ref):
    t = table_ref[...]
    m = jnp.max(t, axis=1)
    s = jnp.sum(jnp.exp(t - m[:, None]), axis=1)
    lse_ref[...] = m + jnp.log(s)


def _lse_call(table):
    return pl.pallas_call(
        _lse_body,
        out_shape=jax.ShapeDtypeStruct((V,), jnp.float32),
    )(table)


def _loss_body(part_ref, out_ref):
    out_ref[0, 0] = jnp.sum(part_ref[...]) * (1.0 / BT)


def _loss_call(part):
    return pl.pallas_call(
        _loss_body,
        out_shape=jax.ShapeDtypeStruct((1, 1), jnp.float32),
        out_specs=pl.BlockSpec(memory_space=pltpu.SMEM),
    )(part)


@functools.partial(
    pl.kernel,
    out_type=(
        jax.ShapeDtypeStruct((NB, 8, BT), jnp.float32),
        jax.ShapeDtypeStruct((NW, 16), jnp.float32),
    ),
    mesh=plsc.VectorSubcoreMesh(core_axis_name="c", subcore_axis_name="s"),
    compiler_params=pltpu.CompilerParams(
        needs_layout_passes=False, use_tc_tiling_on_sc=True
    ),
    scratch_types=[
        pltpu.VMEM((NCOL * V,), jnp.float32),
        pltpu.VMEM((2, S), jnp.int32),
        pltpu.VMEM((2, S), jnp.int32),
        pltpu.VMEM((V,), jnp.float32),
        pltpu.VMEM((BPW,), jnp.int32),
        pltpu.VMEM((2, NCOL, S), jnp.float32),
        pltpu.VMEM((16,), jnp.float32),
        pltpu.SemaphoreType.DMA,
        pltpu.SemaphoreType.DMA,
        pltpu.SemaphoreType.DMA,
        pltpu.SemaphoreType.DMA,
    ],
)
def _sc_gather(tt_hbm, idx_hbm, tgt_hbm, lse_hbm, out_hbm, part_hbm,
               tt_v, idx_v, tgt_v, lse_v, idxl_v, slab_v, acc_v,
               lsem0, lsem1, wsem0, wsem1):
    wid = lax.axis_index("s") * NC + lax.axis_index("c")

    for rep in range(REPS):
        b = wid + NW * rep

        @pl.when(b < NB)
        def _():
            pltpu.sync_copy(tt_hbm.at[pl.ds(b * 8 * V, 8 * V)],
                            tt_v.at[pl.ds(rep * 8 * V, 8 * V)])

    pltpu.sync_copy(lse_hbm, lse_v)

    pltpu.sync_copy(idx_hbm.at[pl.ds(wid * BPW, BPW)], idxl_v)

    def lse_grp(g, a):
        tok = idxl_v[pl.ds(g * 16, 16)]
        return a + plsc.load_gather(lse_v, [tok])

    acc = plsc.parallel_loop(0, BPW // 16, 1, unroll=8,
                             carry=jnp.zeros((16,), jnp.float32))(lse_grp)

    def load_chunk(ci, par, lsem):
        pltpu.async_copy(idx_hbm.at[pl.ds(ci * S, S)], idx_v.at[par], lsem)
        pltpu.async_copy(tgt_hbm.at[pl.ds(ci * S, S)], tgt_v.at[par], lsem)

    def wait_load(par, lsem):
        pltpu.make_async_copy(idx_hbm.at[pl.ds(0, S)], idx_v.at[par],
                              lsem).wait()
        pltpu.make_async_copy(tgt_hbm.at[pl.ds(0, S)], tgt_v.at[par],
                              lsem).wait()

    def start_writes(ci, par, wsem):
        s0 = ci * S
        for rep in range(REPS):
            b = wid + NW * rep

            @pl.when(b < NB)
            def _():
                pltpu.async_copy(slab_v.at[par, pl.ds(rep * 8, 8)],
                                 out_hbm.at[b, :, pl.ds(s0, S)], wsem)

    def wait_writes(par, wsem):
        for rep in range(REPS):
            b = wid + NW * rep

            @pl.when(b < NB)
            def _():
                pltpu.make_async_copy(slab_v.at[par, pl.ds(rep * 8, 8)],
                                      out_hbm.at[b, :, pl.ds(0, S)],
                                      wsem).wait()

    load_chunk(0, 0, lsem0)
    load_chunk(1, 1, lsem1)

    def chunk(ci, acc):
        par = lax.rem(ci, 2)

        @pl.when(par == 0)
        def _():
            wait_load(0, lsem0)

        @pl.when(par == 1)
        def _():
            wait_load(1, lsem1)

        @pl.when((par == 0) & (ci >= 2))
        def _():
            wait_writes(0, wsem0)

        @pl.when((par == 1) & (ci >= 2))
        def _():
            wait_writes(1, wsem1)

        def gi_body(gb, acc):
            tok = idx_v[par, pl.ds(gb, 16)]
            tg = tgt_v[par, pl.ds(gb, 16)]

            def col4(k0):
                for j in range(4):
                    k = k0 + j
                    vals = plsc.load_gather(tt_v, [tok + k * V])
                    slab_v[par, k, pl.ds(gb, 16)] = vals

            plsc.parallel_loop(0, NCOL, 4, unroll=8)(col4)

            for rep in range(REPS):
                rb = 8 * (wid + NW * rep)
                m = (tg >= rb) & (tg < rb + 8)
                pidx = (tg - rb + 8 * rep) * V + tok
                p = plsc.load_gather(tt_v, [pidx], mask=m)
                acc = acc - jnp.where(m, p, 0.0)
            return acc

        acc = plsc.parallel_loop(0, S, 16, unroll=2, carry=acc)(gi_body)

        @pl.when(par == 0)
        def _():
            start_writes(ci, 0, wsem0)

            @pl.when(ci + 2 < NCHUNK)
            def _():
                load_chunk(ci + 2, 0, lsem0)

        @pl.when(par == 1)
        def _():
            start_writes(ci, 1, wsem1)

            @pl.when(ci + 2 < NCHUNK)
            def _():
                load_chunk(ci + 2, 1, lsem1)

        return acc

    acc = lax.fori_loop(0, NCHUNK, chunk, acc)
    wait_writes(0, wsem0)
    wait_writes(1, wsem1)
    acc_v[...] = acc
    pltpu.sync_copy(acc_v, part_hbm.at[wid])


def kernel(idx, targets, table):
    idxf = idx.reshape(-1).astype(jnp.int32)
    tgtf = targets.reshape(-1).astype(jnp.int32)
    tt_flat = jnp.transpose(table).reshape(-1)
    lse = _lse_call(table)
    out3, part = _sc_gather(tt_flat, idxf, tgtf, lse)
    logits = jnp.transpose(out3.reshape(V, BT))
    loss = _loss_call(part)[0, 0]
    return (logits, loss)


# speedup vs baseline: 1.6078x; 1.0276x over previous
import functools

import jax
import jax.numpy as jnp
from jax import lax
from jax.experimental import pallas as pl
from jax.experimental.pallas import tpu as pltpu
from jax.experimental.pallas import tpu_sc as plsc

V = 1000
BT = 51200
NC, NS = 2, 16
NW = NC * NS
NB = 125
REPS = 4
NCOL = REPS * 8
S = 1024
NCHUNK = BT // S
BPW = BT // NW


def _lse_body(table_ref, lse_ref):
    t = table_ref[...]
    m = jnp.max(t, axis=1)
    s = jnp.sum(jnp.exp(t - m[:, None]), axis=1)
    lse_ref[...] = m + jnp.log(s)


def _lse_call(table):
    return pl.pallas_call(
        _lse_body,
        out_shape=jax.ShapeDtypeStruct((V,), jnp.float32),
    )(table)


def _loss_body(part_ref, out_ref):
    out_ref[0, 0] = jnp.sum(part_ref[...]) * (1.0 / BT)


def _loss_call(part):
    return pl.pallas_call(
        _loss_body,
        out_shape=jax.ShapeDtypeStruct((1, 1), jnp.float32),
        out_specs=pl.BlockSpec(memory_space=pltpu.SMEM),
    )(part)


@functools.partial(
    pl.kernel,
    out_type=(
        jax.ShapeDtypeStruct((NB, 8, BT), jnp.float32),
        jax.ShapeDtypeStruct((NW, 16), jnp.float32),
    ),
    mesh=plsc.VectorSubcoreMesh(core_axis_name="c", subcore_axis_name="s"),
    compiler_params=pltpu.CompilerParams(
        needs_layout_passes=False, use_tc_tiling_on_sc=True
    ),
    scratch_types=[
        pltpu.VMEM((NCOL * V,), jnp.float32),
        pltpu.VMEM((2, S), jnp.int32),
        pltpu.VMEM((2, S), jnp.int32),
        pltpu.VMEM((V,), jnp.float32),
        pltpu.VMEM((BPW,), jnp.int32),
        pltpu.VMEM((2, NCOL, S), jnp.float32),
        pltpu.VMEM((16,), jnp.float32),
        pltpu.SemaphoreType.DMA,
        pltpu.SemaphoreType.DMA,
        pltpu.SemaphoreType.DMA,
        pltpu.SemaphoreType.DMA,
    ],
)
def _sc_gather(tt_hbm, idx_hbm, tgt_hbm, lse_hbm, out_hbm, part_hbm,
               tt_v, idx_v, tgt_v, lse_v, idxl_v, slab_v, acc_v,
               lsem0, lsem1, wsem0, wsem1):
    wid = lax.axis_index("s") * NC + lax.axis_index("c")

    for rep in range(REPS):
        b = wid + NW * rep

        @pl.when(b < NB)
        def _():
            pltpu.sync_copy(tt_hbm.at[pl.ds(b * 8 * V, 8 * V)],
                            tt_v.at[pl.ds(rep * 8 * V, 8 * V)])

    pltpu.sync_copy(lse_hbm, lse_v)

    pltpu.sync_copy(idx_hbm.at[pl.ds(wid * BPW, BPW)], idxl_v)

    def lse_grp(g, a):
        tok = idxl_v[pl.ds(g * 16, 16)]
        return a + plsc.load_gather(lse_v, [tok])

    acc = plsc.parallel_loop(0, BPW // 16, 1, unroll=8,
                             carry=jnp.zeros((16,), jnp.float32))(lse_grp)

    def load_chunk(ci, par, lsem):
        pltpu.async_copy(idx_hbm.at[pl.ds(ci * S, S)], idx_v.at[par], lsem)
        pltpu.async_copy(tgt_hbm.at[pl.ds(ci * S, S)], tgt_v.at[par], lsem)

    def wait_load(par, lsem):
        pltpu.make_async_copy(idx_hbm.at[pl.ds(0, S)], idx_v.at[par],
                              lsem).wait()
        pltpu.make_async_copy(tgt_hbm.at[pl.ds(0, S)], tgt_v.at[par],
                              lsem).wait()

    def start_writes(ci, par, wsem):
        s0 = ci * S
        for rep in range(REPS):
            b = wid + NW * rep

            @pl.when(b < NB)
            def _():
                pltpu.async_copy(slab_v.at[par, pl.ds(rep * 8, 8)],
                                 out_hbm.at[b, :, pl.ds(s0, S)], wsem)

    def wait_writes(par, wsem):
        for rep in range(REPS):
            b = wid + NW * rep

            @pl.when(b < NB)
            def _():
                pltpu.make_async_copy(slab_v.at[par, pl.ds(rep * 8, 8)],
                                      out_hbm.at[b, :, pl.ds(0, S)],
                                      wsem).wait()

    load_chunk(0, 0, lsem0)
    load_chunk(1, 1, lsem1)

    def chunk(ci, acc):
        par = lax.rem(ci, 2)

        @pl.when(par == 0)
        def _():
            wait_load(0, lsem0)

        @pl.when(par == 1)
        def _():
            wait_load(1, lsem1)

        @pl.when((par == 0) & (ci >= 2))
        def _():
            wait_writes(0, wsem0)

        @pl.when((par == 1) & (ci >= 2))
        def _():
            wait_writes(1, wsem1)

        def gi_body(gb, acc):
            tok = idx_v[par, pl.ds(gb, 16)]
            tg = tgt_v[par, pl.ds(gb, 16)]

            def col4(k0):
                for j in range(4):
                    k = k0 + j
                    vals = plsc.load_gather(tt_v, [tok + k * V])
                    slab_v[par, k, pl.ds(gb, 16)] = vals

            plsc.parallel_loop(0, NCOL, 4, unroll=8)(col4)

            for rep in range(REPS):
                rb = 8 * (wid + NW * rep)
                m = (tg >= rb) & (tg < rb + 8)
                pidx = (tg - rb + 8 * rep) * V + tok
                p = plsc.load_gather(tt_v, [pidx], mask=m)
                acc = acc - jnp.where(m, p, 0.0)
            return acc

        acc = plsc.parallel_loop(0, S, 16, unroll=4, carry=acc)(gi_body)

        @pl.when(par == 0)
        def _():
            start_writes(ci, 0, wsem0)

            @pl.when(ci + 2 < NCHUNK)
            def _():
                load_chunk(ci + 2, 0, lsem0)

        @pl.when(par == 1)
        def _():
            start_writes(ci, 1, wsem1)

            @pl.when(ci + 2 < NCHUNK)
            def _():
                load_chunk(ci + 2, 1, lsem1)

        return acc

    acc = lax.fori_loop(0, NCHUNK, chunk, acc)
    wait_writes(0, wsem0)
    wait_writes(1, wsem1)
    acc_v[...] = acc
    pltpu.sync_copy(acc_v, part_hbm.at[wid])


def kernel(idx, targets, table):
    idxf = idx.reshape(-1).astype(jnp.int32)
    tgtf = targets.reshape(-1).astype(jnp.int32)
    tt_flat = jnp.transpose(table).reshape(-1)
    lse = _lse_call(table)
    out3, part = _sc_gather(tt_flat, idxf, tgtf, lse)
    logits = jnp.transpose(out3.reshape(V, BT))
    loss = _loss_call(part)[0, 0]
    return (logits, loss)


# speedup vs baseline: 1.6180x; 1.0063x over previous
import functools

import jax
import jax.numpy as jnp
from jax import lax
from jax.experimental import pallas as pl
from jax.experimental.pallas import tpu as pltpu
from jax.experimental.pallas import tpu_sc as plsc

V = 1000
BT = 51200
NC, NS = 2, 16
NW = NC * NS
NB = 125
REPS = 4
NCOL = REPS * 8
S = 1024
NCHUNK = BT // S
BPW = BT // NW


def _lse_body(table_ref, lse_ref):
    t = table_ref[...]
    m = jnp.max(t, axis=1)
    s = jnp.sum(jnp.exp(t - m[:, None]), axis=1)
    lse_ref[...] = m + jnp.log(s)


def _lse_call(table):
    return pl.pallas_call(
        _lse_body,
        out_shape=jax.ShapeDtypeStruct((V,), jnp.float32),
    )(table)


def _loss_body(part_ref, out_ref):
    out_ref[0, 0] = jnp.sum(part_ref[...]) * (1.0 / BT)


def _loss_call(part):
    return pl.pallas_call(
        _loss_body,
        out_shape=jax.ShapeDtypeStruct((1, 1), jnp.float32),
        out_specs=pl.BlockSpec(memory_space=pltpu.SMEM),
    )(part)


@functools.partial(
    pl.kernel,
    out_type=(
        jax.ShapeDtypeStruct((NB, 8, BT), jnp.float32),
        jax.ShapeDtypeStruct((NW, 16), jnp.float32),
    ),
    mesh=plsc.VectorSubcoreMesh(core_axis_name="c", subcore_axis_name="s"),
    compiler_params=pltpu.CompilerParams(
        needs_layout_passes=False, use_tc_tiling_on_sc=True
    ),
    scratch_types=[
        pltpu.VMEM((NCOL * V,), jnp.float32),
        pltpu.VMEM((2, S), jnp.int32),
        pltpu.VMEM((2, S), jnp.int32),
        pltpu.VMEM((V,), jnp.float32),
        pltpu.VMEM((BPW,), jnp.int32),
        pltpu.VMEM((2, NCOL, S), jnp.float32),
        pltpu.VMEM((16,), jnp.float32),
        pltpu.SemaphoreType.DMA,
        pltpu.SemaphoreType.DMA,
        pltpu.SemaphoreType.DMA,
        pltpu.SemaphoreType.DMA,
    ],
)
def _sc_gather(tt_hbm, idx_hbm, tgt_hbm, lse_hbm, out_hbm, part_hbm,
               tt_v, idx_v, tgt_v, lse_v, idxl_v, slab_v, acc_v,
               lsem0, lsem1, wsem0, wsem1):
    wid = lax.axis_index("s") * NC + lax.axis_index("c")

    for rep in range(REPS):
        b = wid + NW * rep

        @pl.when(b < NB)
        def _():
            pltpu.sync_copy(tt_hbm.at[pl.ds(b * 8 * V, 8 * V)],
                            tt_v.at[pl.ds(rep * 8 * V, 8 * V)])

    pltpu.sync_copy(lse_hbm, lse_v)

    pltpu.sync_copy(idx_hbm.at[pl.ds(wid * BPW, BPW)], idxl_v)

    def lse_grp(g, a):
        tok = idxl_v[pl.ds(g * 16, 16)]
        return a + plsc.load_gather(lse_v, [tok])

    acc = plsc.parallel_loop(0, BPW // 16, 1, unroll=8,
                             carry=jnp.zeros((16,), jnp.float32))(lse_grp)

    def load_chunk(ci, par, lsem):
        pltpu.async_copy(idx_hbm.at[pl.ds(ci * S, S)], idx_v.at[par], lsem)
        pltpu.async_copy(tgt_hbm.at[pl.ds(ci * S, S)], tgt_v.at[par], lsem)

    def wait_load(par, lsem):
        pltpu.make_async_copy(idx_hbm.at[pl.ds(0, S)], idx_v.at[par],
                              lsem).wait()
        pltpu.make_async_copy(tgt_hbm.at[pl.ds(0, S)], tgt_v.at[par],
                              lsem).wait()

    def start_writes(ci, par, wsem):
        s0 = ci * S
        for rep in range(REPS):
            b = wid + NW * rep

            @pl.when(b < NB)
            def _():
                pltpu.async_copy(slab_v.at[par, pl.ds(rep * 8, 8)],
                                 out_hbm.at[b, :, pl.ds(s0, S)], wsem)

    def wait_writes(par, wsem):
        for rep in range(REPS):
            b = wid + NW * rep

            @pl.when(b < NB)
            def _():
                pltpu.make_async_copy(slab_v.at[par, pl.ds(rep * 8, 8)],
                                      out_hbm.at[b, :, pl.ds(0, S)],
                                      wsem).wait()

    load_chunk(0, 0, lsem0)
    load_chunk(1, 1, lsem1)

    def chunk(ci, acc):
        par = lax.rem(ci, 2)

        @pl.when(par == 0)
        def _():
            wait_load(0, lsem0)

        @pl.when(par == 1)
        def _():
            wait_load(1, lsem1)

        @pl.when((par == 0) & (ci >= 2))
        def _():
            wait_writes(0, wsem0)

        @pl.when((par == 1) & (ci >= 2))
        def _():
            wait_writes(1, wsem1)

        def gi_body(gb, acc):
            tok = idx_v[par, pl.ds(gb, 16)]
            tg = tgt_v[par, pl.ds(gb, 16)]

            def col4(k0):
                for j in range(4):
                    k = k0 + j
                    vals = plsc.load_gather(tt_v, [tok + k * V])
                    slab_v[par, k, pl.ds(gb, 16)] = vals

            plsc.parallel_loop(0, NCOL, 4, unroll=8)(col4)

            for rep in range(REPS):
                rb = 8 * (wid + NW * rep)
                m = (tg >= rb) & (tg < rb + 8)
                pidx = (tg - rb + 8 * rep) * V + tok
                p = plsc.load_gather(tt_v, [pidx], mask=m)
                acc = acc - jnp.where(m, p, 0.0)
            return acc

        acc = plsc.parallel_loop(0, S, 16, unroll=8, carry=acc)(gi_body)

        @pl.when(par == 0)
        def _():
            start_writes(ci, 0, wsem0)

            @pl.when(ci + 2 < NCHUNK)
            def _():
                load_chunk(ci + 2, 0, lsem0)

        @pl.when(par == 1)
        def _():
            start_writes(ci, 1, wsem1)

            @pl.when(ci + 2 < NCHUNK)
            def _():
                load_chunk(ci + 2, 1, lsem1)

        return acc

    acc = lax.fori_loop(0, NCHUNK, chunk, acc)
    wait_writes(0, wsem0)
    wait_writes(1, wsem1)
    acc_v[...] = acc
    pltpu.sync_copy(acc_v, part_hbm.at[wid])


def kernel(idx, targets, table):
    idxf = idx.reshape(-1).astype(jnp.int32)
    tgtf = targets.reshape(-1).astype(jnp.int32)
    tt_flat = jnp.transpose(table).reshape(-1)
    lse = _lse_call(table)
    out3, part = _sc_gather(tt_flat, idxf, tgtf, lse)
    logits = jnp.transpose(out3.reshape(V, BT))
    loss = _loss_call(part)[0, 0]
    return (logits, loss)


# speedup vs baseline: 1.6496x; 1.0195x over previous
import functools

import jax
import jax.numpy as jnp
from jax import lax
from jax.experimental import pallas as pl
from jax.experimental.pallas import tpu as pltpu
from jax.experimental.pallas import tpu_sc as plsc

V = 1000
BT = 51200
NC, NS = 2, 16
NW = NC * NS
NB = 125
REPS = 4
NCOL = REPS * 8
S = 1280
NCHUNK = BT // S
BPW = BT // NW


def _lse_body(table_ref, lse_ref):
    t = table_ref[...]
    m = jnp.max(t, axis=1)
    s = jnp.sum(jnp.exp(t - m[:, None]), axis=1)
    lse_ref[...] = m + jnp.log(s)


def _lse_call(table):
    return pl.pallas_call(
        _lse_body,
        out_shape=jax.ShapeDtypeStruct((V,), jnp.float32),
    )(table)


def _loss_body(part_ref, out_ref):
    out_ref[0, 0] = jnp.sum(part_ref[...]) * (1.0 / BT)


def _loss_call(part):
    return pl.pallas_call(
        _loss_body,
        out_shape=jax.ShapeDtypeStruct((1, 1), jnp.float32),
        out_specs=pl.BlockSpec(memory_space=pltpu.SMEM),
    )(part)


@functools.partial(
    pl.kernel,
    out_type=(
        jax.ShapeDtypeStruct((NB, 8, BT), jnp.float32),
        jax.ShapeDtypeStruct((NW, 16), jnp.float32),
    ),
    mesh=plsc.VectorSubcoreMesh(core_axis_name="c", subcore_axis_name="s"),
    compiler_params=pltpu.CompilerParams(
        needs_layout_passes=False, use_tc_tiling_on_sc=True
    ),
    scratch_types=[
        pltpu.VMEM((NCOL * V,), jnp.float32),
        pltpu.VMEM((2, S), jnp.int32),
        pltpu.VMEM((2, S), jnp.int32),
        pltpu.VMEM((V,), jnp.float32),
        pltpu.VMEM((BPW,), jnp.int32),
        pltpu.VMEM((2, NCOL, S), jnp.float32),
        pltpu.VMEM((16,), jnp.float32),
        pltpu.SemaphoreType.DMA,
        pltpu.SemaphoreType.DMA,
        pltpu.SemaphoreType.DMA,
        pltpu.SemaphoreType.DMA,
    ],
)
def _sc_gather(tt_hbm, idx_hbm, tgt_hbm, lse_hbm, out_hbm, part_hbm,
               tt_v, idx_v, tgt_v, lse_v, idxl_v, slab_v, acc_v,
               lsem0, lsem1, wsem0, wsem1):
    wid = lax.axis_index("s") * NC + lax.axis_index("c")

    for rep in range(REPS):
        b = wid + NW * rep

        @pl.when(b < NB)
        def _():
            pltpu.sync_copy(tt_hbm.at[pl.ds(b * 8 * V, 8 * V)],
                            tt_v.at[pl.ds(rep * 8 * V, 8 * V)])

    pltpu.sync_copy(lse_hbm, lse_v)

    pltpu.sync_copy(idx_hbm.at[pl.ds(wid * BPW, BPW)], idxl_v)

    def lse_grp(g, a):
        tok = idxl_v[pl.ds(g * 16, 16)]
        return a + plsc.load_gather(lse_v, [tok])

    acc = plsc.parallel_loop(0, BPW // 16, 1, unroll=8,
                             carry=jnp.zeros((16,), jnp.float32))(lse_grp)

    def load_chunk(ci, par, lsem):
        pltpu.async_copy(idx_hbm.at[pl.ds(ci * S, S)], idx_v.at[par], lsem)
        pltpu.async_copy(tgt_hbm.at[pl.ds(ci * S, S)], tgt_v.at[par], lsem)

    def wait_load(par, lsem):
        pltpu.make_async_copy(idx_hbm.at[pl.ds(0, S)], idx_v.at[par],
                              lsem).wait()
        pltpu.make_async_copy(tgt_hbm.at[pl.ds(0, S)], tgt_v.at[par],
                              lsem).wait()

    def start_writes(ci, par, wsem):
        s0 = ci * S
        for rep in range(REPS):
            b = wid + NW * rep

            @pl.when(b < NB)
            def _():
                pltpu.async_copy(slab_v.at[par, pl.ds(rep * 8, 8)],
                                 out_hbm.at[b, :, pl.ds(s0, S)], wsem)

    def wait_writes(par, wsem):
        for rep in range(REPS):
            b = wid + NW * rep

            @pl.when(b < NB)
            def _():
                pltpu.make_async_copy(slab_v.at[par, pl.ds(rep * 8, 8)],
                                      out_hbm.at[b, :, pl.ds(0, S)],
                                      wsem).wait()

    load_chunk(0, 0, lsem0)
    load_chunk(1, 1, lsem1)

    def chunk(ci, acc):
        par = lax.rem(ci, 2)

        @pl.when(par == 0)
        def _():
            wait_load(0, lsem0)

        @pl.when(par == 1)
        def _():
            wait_load(1, lsem1)

        @pl.when((par == 0) & (ci >= 2))
        def _():
            wait_writes(0, wsem0)

        @pl.when((par == 1) & (ci >= 2))
        def _():
            wait_writes(1, wsem1)

        def gi_body(gb, acc):
            tok = idx_v[par, pl.ds(gb, 16)]
            tg = tgt_v[par, pl.ds(gb, 16)]

            def col4(k0):
                for j in range(4):
                    k = k0 + j
                    vals = plsc.load_gather(tt_v, [tok + k * V])
                    slab_v[par, k, pl.ds(gb, 16)] = vals

            plsc.parallel_loop(0, NCOL, 4, unroll=8)(col4)

            for rep in range(REPS):
                rb = 8 * (wid + NW * rep)
                m = (tg >= rb) & (tg < rb + 8)
                pidx = (tg - rb + 8 * rep) * V + tok
                p = plsc.load_gather(tt_v, [pidx], mask=m)
                acc = acc - jnp.where(m, p, 0.0)
            return acc

        acc = plsc.parallel_loop(0, S, 16, unroll=8, carry=acc)(gi_body)

        @pl.when(par == 0)
        def _():
            start_writes(ci, 0, wsem0)

            @pl.when(ci + 2 < NCHUNK)
            def _():
                load_chunk(ci + 2, 0, lsem0)

        @pl.when(par == 1)
        def _():
            start_writes(ci, 1, wsem1)

            @pl.when(ci + 2 < NCHUNK)
            def _():
                load_chunk(ci + 2, 1, lsem1)

        return acc

    acc = lax.fori_loop(0, NCHUNK, chunk, acc)
    wait_writes(0, wsem0)
    wait_writes(1, wsem1)
    acc_v[...] = acc
    pltpu.sync_copy(acc_v, part_hbm.at[wid])


def kernel(idx, targets, table):
    idxf = idx.reshape(-1).astype(jnp.int32)
    tgtf = targets.reshape(-1).astype(jnp.int32)
    tt_flat = jnp.transpose(table).reshape(-1)
    lse = _lse_call(table)
    out3, part = _sc_gather(tt_flat, idxf, tgtf, lse)
    logits = jnp.transpose(out3.reshape(V, BT))
    loss = _loss_call(part)[0, 0]
    return (logits, loss)
